# Initial kernel scaffold; baseline (speedup 1.0000x reference)
import jax
import jax.numpy as jnp
from jax.experimental import pallas as pl


def kernel(sample, map_gender, map_age, map_occupation, user_id_emb, gender_emb, age_emb, occupation_emb, W, b):
    raise NotImplementedError("write your pallas kernel here")


# speedup vs baseline: 1.0499x; 1.0499x over previous
import functools

import jax
import jax.numpy as jnp
from jax import lax
from jax.experimental import pallas as pl
from jax.experimental.pallas import tpu as pltpu
from jax.experimental.pallas import tpu_sc as plsc

_NC = 2
_NS = 16
_NW = _NC * _NS

_USER_DIM = 64
_GENDER_DIM = 8
_AGE_DIM = 16
_OCC_DIM = 32
_FINAL = 128


def _sc_gather(sample, map_gender, map_age, map_occupation,
               user_id_emb, gender_emb, age_emb, occupation_emb):
    batch = sample.shape[0]
    bpw = batch // _NW

    mesh = plsc.VectorSubcoreMesh(core_axis_name="c", subcore_axis_name="s")
    out_type = (
        jax.ShapeDtypeStruct((batch, _USER_DIM), jnp.float32),
        jax.ShapeDtypeStruct((batch, _GENDER_DIM), jnp.float32),
        jax.ShapeDtypeStruct((batch, _AGE_DIM), jnp.float32),
        jax.ShapeDtypeStruct((batch, _OCC_DIM), jnp.float32),
    )

    @functools.partial(
        pl.kernel,
        out_type=out_type,
        mesh=mesh,
        compiler_params=pltpu.CompilerParams(use_tc_tiling_on_sc=False),
        scratch_types=[
            pltpu.VMEM((bpw,), jnp.int32),
            pltpu.VMEM((bpw,), jnp.int32),
            pltpu.VMEM((bpw,), jnp.int32),
            pltpu.VMEM((bpw,), jnp.int32),
            pltpu.VMEM((bpw, _USER_DIM), jnp.float32),
            pltpu.VMEM((bpw, _GENDER_DIM), jnp.float32),
            pltpu.VMEM((bpw, _AGE_DIM), jnp.float32),
            pltpu.VMEM((bpw, _OCC_DIM), jnp.float32),
            pltpu.SemaphoreType.DMA,
            pltpu.SemaphoreType.DMA,
            pltpu.SemaphoreType.DMA,
            pltpu.SemaphoreType.DMA,
        ],
    )
    def gather_kernel(sample_h, mg_h, ma_h, mo_h, ue_h, ge_h, ae_h, oe_h,
                      fu_o, fg_o, fa_o, fo_o,
                      idx_v, gid_v, aid_v, oid_v, fu_v, fg_v, fa_v, fo_v,
                      sem_u, sem_g, sem_a, sem_o):
        wid = lax.axis_index("s") * _NC + lax.axis_index("c")
        base = wid * bpw
        pltpu.sync_copy(sample_h.at[pl.ds(base, bpw)], idx_v)
        cp_u = pltpu.async_copy(ue_h.at[idx_v], fu_v, sem_u)
        cp_g = pltpu.async_copy(mg_h.at[idx_v], gid_v, sem_g)
        cp_a = pltpu.async_copy(ma_h.at[idx_v], aid_v, sem_a)
        cp_o = pltpu.async_copy(mo_h.at[idx_v], oid_v, sem_o)
        cp_g.wait()
        cp_g2 = pltpu.async_copy(ge_h.at[gid_v], fg_v, sem_g)
        cp_a.wait()
        cp_a2 = pltpu.async_copy(ae_h.at[aid_v], fa_v, sem_a)
        cp_o.wait()
        cp_o2 = pltpu.async_copy(oe_h.at[oid_v], fo_v, sem_o)
        cp_u.wait()
        pltpu.sync_copy(fu_v, fu_o.at[pl.ds(base, bpw)])
        cp_g2.wait()
        pltpu.sync_copy(fg_v, fg_o.at[pl.ds(base, bpw)])
        cp_a2.wait()
        pltpu.sync_copy(fa_v, fa_o.at[pl.ds(base, bpw)])
        cp_o2.wait()
        pltpu.sync_copy(fo_v, fo_o.at[pl.ds(base, bpw)])

    return gather_kernel(sample, map_gender, map_age, map_occupation,
                         user_id_emb, gender_emb, age_emb, occupation_emb)


def _tc_mlp(fu, fg, fa, fo, wu, wg, wa, wo, b2):
    batch = fu.shape[0]
    bm = 512

    def body(fu_r, fg_r, fa_r, fo_r, wu_r, wg_r, wa_r, wo_r, b_r, o_r):
        acc = jnp.dot(fu_r[...], wu_r[...], preferred_element_type=jnp.float32)
        acc += jnp.dot(fg_r[...], wg_r[...], preferred_element_type=jnp.float32)
        acc += jnp.dot(fa_r[...], wa_r[...], preferred_element_type=jnp.float32)
        acc += jnp.dot(fo_r[...], wo_r[...], preferred_element_type=jnp.float32)
        o_r[...] = jnp.maximum(acc + b_r[...], 0.0)

    return pl.pallas_call(
        body,
        grid=(batch // bm,),
        in_specs=[
            pl.BlockSpec((bm, _USER_DIM), lambda i: (i, 0)),
            pl.BlockSpec((bm, _GENDER_DIM), lambda i: (i, 0)),
            pl.BlockSpec((bm, _AGE_DIM), lambda i: (i, 0)),
            pl.BlockSpec((bm, _OCC_DIM), lambda i: (i, 0)),
            pl.BlockSpec((_USER_DIM, _FINAL), lambda i: (0, 0)),
            pl.BlockSpec((_GENDER_DIM, _FINAL), lambda i: (0, 0)),
            pl.BlockSpec((_AGE_DIM, _FINAL), lambda i: (0, 0)),
            pl.BlockSpec((_OCC_DIM, _FINAL), lambda i: (0, 0)),
            pl.BlockSpec((1, _FINAL), lambda i: (0, 0)),
        ],
        out_specs=pl.BlockSpec((bm, _FINAL), lambda i: (i, 0)),
        out_shape=jax.ShapeDtypeStruct((batch, _FINAL), jnp.float32),
    )(fu, fg, fa, fo, wu, wg, wa, wo, b2)


def kernel(sample, map_gender, map_age, map_occupation, user_id_emb,
           gender_emb, age_emb, occupation_emb, W, b):
    fu, fg, fa, fo = _sc_gather(sample, map_gender, map_age, map_occupation,
                                user_id_emb, gender_emb, age_emb,
                                occupation_emb)
    wu = W[:_USER_DIM]
    wg = W[_USER_DIM:_USER_DIM + _GENDER_DIM]
    wa = W[_USER_DIM + _GENDER_DIM:_USER_DIM + _GENDER_DIM + _AGE_DIM]
    wo = W[_USER_DIM + _GENDER_DIM + _AGE_DIM:]
    return _tc_mlp(fu, fg, fa, fo, wu, wg, wa, wo, b.reshape(1, _FINAL))


# speedup vs baseline: 1.1670x; 1.1116x over previous
import functools

import jax
import jax.numpy as jnp
from jax import lax
from jax.experimental import pallas as pl
from jax.experimental.pallas import tpu as pltpu
from jax.experimental.pallas import tpu_sc as plsc

_NC = 2
_NS = 16
_NW = _NC * _NS

_USER_DIM = 64
_GENDER_DIM = 8
_AGE_DIM = 16
_OCC_DIM = 32
_IN_SIZE = _USER_DIM + _GENDER_DIM + _AGE_DIM + _OCC_DIM
_FINAL = 128


def _sc_gather(sample, map_gender, map_age, map_occupation,
               user_id_emb, gender_emb, age_emb, occupation_emb):
    batch = sample.shape[0]
    bpw = batch // _NW

    mesh = plsc.VectorSubcoreMesh(core_axis_name="c", subcore_axis_name="s")

    @functools.partial(
        pl.kernel,
        out_type=jax.ShapeDtypeStruct((batch, _FINAL), jnp.float32),
        mesh=mesh,
        compiler_params=pltpu.CompilerParams(use_tc_tiling_on_sc=False),
        scratch_types=[
            pltpu.VMEM((bpw,), jnp.int32),
            pltpu.VMEM((bpw,), jnp.int32),
            pltpu.VMEM((bpw,), jnp.int32),
            pltpu.VMEM((bpw,), jnp.int32),
            pltpu.VMEM((bpw, _USER_DIM), jnp.float32),
            pltpu.VMEM((bpw, _GENDER_DIM), jnp.float32),
            pltpu.VMEM((bpw, _AGE_DIM), jnp.float32),
            pltpu.VMEM((bpw, _OCC_DIM), jnp.float32),
            pltpu.SemaphoreType.DMA,
            pltpu.SemaphoreType.DMA,
            pltpu.SemaphoreType.DMA,
            pltpu.SemaphoreType.DMA,
        ],
    )
    def gather_kernel(sample_h, mg_h, ma_h, mo_h, ue_h, ge_h, ae_h, oe_h,
                      feats_o,
                      idx_v, gid_v, aid_v, oid_v, fu_v, fg_v, fa_v, fo_v,
                      sem_u, sem_g, sem_a, sem_o):
        wid = lax.axis_index("s") * _NC + lax.axis_index("c")
        base = wid * bpw
        pltpu.sync_copy(sample_h.at[pl.ds(base, bpw)], idx_v)
        cp_u = pltpu.async_copy(ue_h.at[idx_v], fu_v, sem_u)
        cp_g = pltpu.async_copy(mg_h.at[idx_v], gid_v, sem_g)
        cp_a = pltpu.async_copy(ma_h.at[idx_v], aid_v, sem_a)
        cp_o = pltpu.async_copy(mo_h.at[idx_v], oid_v, sem_o)
        cp_g.wait()
        cp_g2 = pltpu.async_copy(ge_h.at[gid_v], fg_v, sem_g)
        cp_a.wait()
        cp_a2 = pltpu.async_copy(ae_h.at[aid_v], fa_v, sem_a)
        cp_o.wait()
        cp_o2 = pltpu.async_copy(oe_h.at[oid_v], fo_v, sem_o)
        cp_u.wait()
        pltpu.sync_copy(fu_v, feats_o.at[pl.ds(base, bpw), pl.ds(0, _USER_DIM)])
        cp_g2.wait()
        pltpu.sync_copy(
            fg_v, feats_o.at[pl.ds(base, bpw), pl.ds(_USER_DIM, _GENDER_DIM)])
        cp_a2.wait()
        pltpu.sync_copy(fa_v, feats_o.at[pl.ds(base, bpw), pl.ds(72, _AGE_DIM)])
        cp_o2.wait()
        pltpu.sync_copy(fo_v, feats_o.at[pl.ds(base, bpw), pl.ds(88, _OCC_DIM)])

    return gather_kernel(sample, map_gender, map_age, map_occupation,
                         user_id_emb, gender_emb, age_emb, occupation_emb)


def _tc_mlp(feats, W, b):
    batch = feats.shape[0]
    bm = 1024

    def body(feats_r, w_r, b_r, o_r):
        x = feats_r[...][:, :_IN_SIZE]
        acc = jnp.dot(x, w_r[...], preferred_element_type=jnp.float32)
        o_r[...] = jnp.maximum(acc + b_r[...].reshape(1, _FINAL), 0.0)

    return pl.pallas_call(
        body,
        grid=(batch // bm,),
        in_specs=[
            pl.BlockSpec((bm, _FINAL), lambda i: (i, 0)),
            pl.BlockSpec((_IN_SIZE, _FINAL), lambda i: (0, 0)),
            pl.BlockSpec((_FINAL,), lambda i: (0,)),
        ],
        out_specs=pl.BlockSpec((bm, _FINAL), lambda i: (i, 0)),
        out_shape=jax.ShapeDtypeStruct((batch, _FINAL), jnp.float32),
    )(feats, W, b)


def kernel(sample, map_gender, map_age, map_occupation, user_id_emb,
           gender_emb, age_emb, occupation_emb, W, b):
    feats = _sc_gather(sample, map_gender, map_age, map_occupation,
                       user_id_emb, gender_emb, age_emb, occupation_emb)
    return _tc_mlp(feats, W, b)
